# Initial kernel scaffold; baseline (speedup 1.0000x reference)
import jax
import jax.numpy as jnp
from jax.experimental import pallas as pl


def kernel(x, neighbours, enc_W, enc_b, decoder, bandwidth_layers):
    raise NotImplementedError("write your pallas kernel here")


# speedup vs baseline: 2.1981x; 2.1981x over previous
import functools

import jax
import jax.numpy as jnp
from jax import lax
from jax.experimental import pallas as pl
from jax.experimental.pallas import tpu as pltpu
from jax.experimental.pallas import tpu_sc as plsc

N_NODES = 10000
N_LAT = 32
MU = 16
N_BATCH = 4

BLK = 1000
LANES = MU * N_LAT

_SC_WORKERS = 32
_SC_ROWS = N_NODES * MU // _SC_WORKERS
_SC_CHUNK = 1000


def _encode_body(x_ref, w_ref, b_ref, out_ref):
    enc = lax.dot_general(x_ref[...], w_ref[...],
                          (((1,), (1,)), ((), ())),
                          preferred_element_type=jnp.float32)
    out_ref[...] = enc + b_ref[...]


def _encode(x, enc_W, enc_b2):
    return pl.pallas_call(
        _encode_body,
        out_shape=jax.ShapeDtypeStruct((N_BATCH, N_LAT), jnp.float32),
    )(x, enc_W, enc_b2)


def _bw_body(enc_ref, l_ref, sig_ref):
    l_i = l_ref[0]
    logit_t = lax.dot_general(l_i, enc_ref[...],
                              (((0,), (1,)), ((), ())),
                              preferred_element_type=jnp.float32)
    sig_ref[0] = jax.nn.sigmoid(logit_t) * MU


def _bandwidths(encoded, bandwidth_layers):
    return pl.pallas_call(
        _bw_body,
        grid=(N_LAT,),
        in_specs=[
            pl.BlockSpec((N_BATCH, N_LAT), lambda i: (0, 0)),
            pl.BlockSpec((1, N_LAT, N_NODES), lambda i: (i, 0, 0)),
        ],
        out_specs=pl.BlockSpec((1, N_NODES, N_BATCH), lambda i: (i, 0, 0)),
        out_shape=jax.ShapeDtypeStruct((N_LAT, N_NODES, N_BATCH),
                                       jnp.float32),
    )(encoded, bandwidth_layers)


def _gather_body(table_hbm, idx_hbm, out_hbm, idx_v, rows_v, sem):
    wid = lax.axis_index("s") * 2 + lax.axis_index("c")
    base = wid * _SC_ROWS
    for c in range(_SC_ROWS // _SC_CHUNK):
        off = base + c * _SC_CHUNK
        pltpu.sync_copy(idx_hbm.at[pl.ds(off, _SC_CHUNK)], idx_v)
        pltpu.async_copy(table_hbm.at[idx_v], rows_v, sem).wait()
        pltpu.sync_copy(rows_v, out_hbm.at[pl.ds(off, _SC_CHUNK)])


def _gather(table, idx):
    mesh = plsc.VectorSubcoreMesh(core_axis_name="c", subcore_axis_name="s")
    kfn = functools.partial(
        pl.kernel, mesh=mesh,
        out_type=jax.ShapeDtypeStruct((N_NODES * MU, N_LAT), jnp.float32),
        scratch_types=[
            pltpu.VMEM((_SC_CHUNK,), jnp.int32),
            pltpu.VMEM((_SC_CHUNK, N_LAT), jnp.float32),
            pltpu.SemaphoreType.DMA,
        ],
        compiler_params=pltpu.CompilerParams(use_tc_tiling_on_sc=False),
    )(_gather_body)
    return kfn(table, idx)


def _gred(x):
    x = x[:, :256] + x[:, 256:]
    x = x[:, :128] + x[:, 128:]
    x = x[:, :64] + x[:, 64:]
    return x[:, :32] + x[:, 32:]


def _main_body(enc_ref, sig_ref, g_ref, out_ref):
    E = enc_ref[...]
    G = g_ref[...]

    s128 = jnp.concatenate([sig_ref[i] for i in range(N_LAT)], axis=1)
    r_iota = lax.broadcasted_iota(jnp.int32, (128, 128), 0)
    c_iota = lax.broadcasted_iota(jnp.int32, (128, 128), 1)
    perm = jnp.where(c_iota == (r_iota % N_BATCH) * N_LAT
                     + r_iota // N_BATCH, 1.0, 0.0)
    sp = lax.dot_general(s128, perm, (((1,), (0,)), ((), ())),
                         preferred_element_type=jnp.float32)

    jv = (lax.broadcasted_iota(jnp.int32, (BLK, LANES), 1) // N_LAT
          ).astype(jnp.float32)
    jsq = jv * jv

    cols = []
    for b in range(N_BATCH):
        s = sp[:, b * N_LAT:(b + 1) * N_LAT]
        srep = jnp.concatenate([s] * MU, axis=1)
        win = jnp.maximum(1.0 - jsq / (srep * srep), 0.0)
        num = _gred(win * G)
        den = _gred(win)
        contrib = (num / den) * E[b:b + 1, :]
        cols.append(jnp.sum(contrib, axis=1, keepdims=True))
    out_ref[...] = jnp.concatenate(cols, axis=1)


def _main(encoded, sig, g2):
    return pl.pallas_call(
        _main_body,
        grid=(N_NODES // BLK,),
        in_specs=[
            pl.BlockSpec((N_BATCH, N_LAT), lambda j: (0, 0)),
            pl.BlockSpec((N_LAT, BLK, N_BATCH), lambda j: (0, j, 0)),
            pl.BlockSpec((BLK, LANES), lambda j: (j, 0)),
        ],
        out_specs=pl.BlockSpec((BLK, N_BATCH), lambda j: (j, 0)),
        out_shape=jax.ShapeDtypeStruct((N_NODES, N_BATCH), jnp.float32),
    )(encoded, sig, g2)


def kernel(x, neighbours, enc_W, enc_b, decoder, bandwidth_layers):
    encoded = _encode(x, enc_W, enc_b.reshape(1, N_LAT))
    sig = _bandwidths(encoded, bandwidth_layers)
    table = decoder.T
    idx = neighbours.reshape(N_NODES * MU)
    gathered = _gather(table, idx)
    g2 = gathered.reshape(N_NODES, LANES)
    out_t = _main(encoded, sig, g2)
    return out_t.T


# speedup vs baseline: 3.4539x; 1.5713x over previous
import functools

import jax
import jax.numpy as jnp
from jax import lax
from jax.experimental import pallas as pl
from jax.experimental.pallas import tpu as pltpu
from jax.experimental.pallas import tpu_sc as plsc

N_NODES = 10000
N_LAT = 32
MU = 16
N_BATCH = 4

BLK = 2000
LANES = MU * N_LAT

_SC_WORKERS = 32
_SC_ROWS = N_NODES * MU // _SC_WORKERS
_SC_CHUNK = 1000


def _encode_body(x_ref, w_ref, b_ref, out_ref):
    enc = lax.dot_general(x_ref[...], w_ref[...],
                          (((1,), (1,)), ((), ())),
                          preferred_element_type=jnp.float32)
    out_ref[...] = enc + b_ref[...]


def _encode(x, enc_W, enc_b2):
    return pl.pallas_call(
        _encode_body,
        out_shape=jax.ShapeDtypeStruct((N_BATCH, N_LAT), jnp.float32),
    )(x, enc_W, enc_b2)


_ICHUNK = 8


def _bw_body(enc_ref, l_ref, sig_ref):
    E = enc_ref[...]
    l8 = l_ref[...].reshape(_ICHUNK * N_LAT, N_NODES)
    nrow = _ICHUNK * N_BATCH
    ncol = _ICHUNK * N_LAT
    erep = jnp.concatenate([E] * _ICHUNK, axis=0)
    erep = jnp.concatenate([erep] * _ICHUNK, axis=1)
    r_iota = lax.broadcasted_iota(jnp.int32, (nrow, ncol), 0)
    c_iota = lax.broadcasted_iota(jnp.int32, (nrow, ncol), 1)
    ebig = jnp.where(r_iota // N_BATCH == c_iota // N_LAT, erep, 0.0)
    logit = lax.dot_general(ebig, l8, (((1,), (0,)), ((), ())),
                            preferred_element_type=jnp.float32)
    s = jax.nn.sigmoid(logit) * MU
    sig_ref[0] = jnp.transpose(s, (1, 0))


def _bandwidths(encoded, bandwidth_layers):
    return pl.pallas_call(
        _bw_body,
        grid=(N_LAT // _ICHUNK,),
        in_specs=[
            pl.BlockSpec((N_BATCH, N_LAT), lambda i: (0, 0)),
            pl.BlockSpec((_ICHUNK, N_LAT, N_NODES), lambda i: (i, 0, 0)),
        ],
        out_specs=pl.BlockSpec((1, N_NODES, _ICHUNK * N_BATCH),
                               lambda i: (i, 0, 0)),
        out_shape=jax.ShapeDtypeStruct((N_LAT // _ICHUNK, N_NODES,
                                        _ICHUNK * N_BATCH), jnp.float32),
    )(encoded, bandwidth_layers)


def _gather_body(table_hbm, idx_hbm, out_hbm, idx_v, rows_v, sem):
    wid = lax.axis_index("s") * 2 + lax.axis_index("c")
    base = wid * _SC_ROWS
    for c in range(_SC_ROWS // _SC_CHUNK):
        off = base + c * _SC_CHUNK
        pltpu.sync_copy(idx_hbm.at[pl.ds(off, _SC_CHUNK)], idx_v)
        pltpu.async_copy(table_hbm.at[idx_v], rows_v, sem).wait()
        pltpu.sync_copy(rows_v, out_hbm.at[pl.ds(off, _SC_CHUNK)])


def _gather(table, idx):
    mesh = plsc.VectorSubcoreMesh(core_axis_name="c", subcore_axis_name="s")
    kfn = functools.partial(
        pl.kernel, mesh=mesh,
        out_type=jax.ShapeDtypeStruct((N_NODES * MU, N_LAT), jnp.float32),
        scratch_types=[
            pltpu.VMEM((_SC_CHUNK,), jnp.int32),
            pltpu.VMEM((_SC_CHUNK, N_LAT), jnp.float32),
            pltpu.SemaphoreType.DMA,
        ],
        compiler_params=pltpu.CompilerParams(use_tc_tiling_on_sc=False),
    )(_gather_body)
    return kfn(table, idx)


def _gred(x):
    x = x[:, :256] + x[:, 256:]
    x = x[:, :128] + x[:, 128:]
    x = x[:, :64] + x[:, 64:]
    return x[:, :32] + x[:, 32:]


def _main_body(enc_ref, sig_ref, g_ref, out_ref):
    E = enc_ref[...]
    G = g_ref[...]

    nchunk = N_LAT // _ICHUNK
    s128 = jnp.concatenate([sig_ref[c] for c in range(nchunk)], axis=1)
    r_iota = lax.broadcasted_iota(jnp.int32, (128, 128), 0)
    c_iota = lax.broadcasted_iota(jnp.int32, (128, 128), 1)
    perm = jnp.where(c_iota == (r_iota % N_BATCH) * N_LAT
                     + _ICHUNK * (r_iota // 32) + (r_iota // N_BATCH) % _ICHUNK,
                     1.0, 0.0)
    sp = lax.dot_general(s128, perm, (((1,), (0,)), ((), ())),
                         preferred_element_type=jnp.float32)

    jv = (lax.broadcasted_iota(jnp.int32, (BLK, LANES), 1) // N_LAT
          ).astype(jnp.float32)
    jsq = jv * jv

    cols = []
    for b in range(N_BATCH):
        s = sp[:, b * N_LAT:(b + 1) * N_LAT]
        srep = jnp.concatenate([s] * MU, axis=1)
        win = jnp.maximum(1.0 - jsq / (srep * srep), 0.0)
        num = _gred(win * G)
        den = _gred(win)
        contrib = (num / den) * E[b:b + 1, :]
        cols.append(jnp.sum(contrib, axis=1, keepdims=True))
    out_ref[...] = jnp.concatenate(cols, axis=1)


def _main(encoded, sig, g2):
    return pl.pallas_call(
        _main_body,
        grid=(N_NODES // BLK,),
        in_specs=[
            pl.BlockSpec((N_BATCH, N_LAT), lambda j: (0, 0)),
            pl.BlockSpec((N_LAT // _ICHUNK, BLK, _ICHUNK * N_BATCH),
                         lambda j: (0, j, 0)),
            pl.BlockSpec((BLK, LANES), lambda j: (j, 0)),
        ],
        out_specs=pl.BlockSpec((BLK, N_BATCH), lambda j: (j, 0)),
        out_shape=jax.ShapeDtypeStruct((N_NODES, N_BATCH), jnp.float32),
    )(encoded, sig, g2)


def kernel(x, neighbours, enc_W, enc_b, decoder, bandwidth_layers):
    encoded = _encode(x, enc_W, enc_b.reshape(1, N_LAT))
    sig = _bandwidths(encoded, bandwidth_layers)
    table = decoder.T
    idx = neighbours.reshape(N_NODES * MU)
    gathered = _gather(table, idx)
    g2 = gathered.reshape(N_NODES, LANES)
    out_t = _main(encoded, sig, g2)
    return out_t.T


# speedup vs baseline: 3.5863x; 1.0383x over previous
import functools

import jax
import jax.numpy as jnp
from jax import lax
from jax.experimental import pallas as pl
from jax.experimental.pallas import tpu as pltpu
from jax.experimental.pallas import tpu_sc as plsc

N_NODES = 10000
N_LAT = 32
MU = 16
N_BATCH = 4

BLK = 2000
LANES = MU * N_LAT

_SC_WORKERS = 32
_SC_ROWS = N_NODES * MU // _SC_WORKERS
_SC_CHUNK = 1000


def _encode_body(x_ref, w_ref, b_ref, out_ref):
    enc = lax.dot_general(x_ref[...], w_ref[...],
                          (((1,), (1,)), ((), ())),
                          preferred_element_type=jnp.float32)
    out_ref[...] = enc + b_ref[...]


def _encode(x, enc_W, enc_b2):
    return pl.pallas_call(
        _encode_body,
        out_shape=jax.ShapeDtypeStruct((N_BATCH, N_LAT), jnp.float32),
    )(x, enc_W, enc_b2)


_ICHUNK = 8


def _bw_body(enc_ref, l_ref, sig_ref):
    E = enc_ref[...]
    l8 = l_ref[...].reshape(_ICHUNK * N_LAT, N_NODES)
    nrow = _ICHUNK * N_BATCH
    ncol = _ICHUNK * N_LAT
    erep = jnp.concatenate([E] * _ICHUNK, axis=0)
    erep = jnp.concatenate([erep] * _ICHUNK, axis=1)
    r_iota = lax.broadcasted_iota(jnp.int32, (nrow, ncol), 0)
    c_iota = lax.broadcasted_iota(jnp.int32, (nrow, ncol), 1)
    ebig = jnp.where(r_iota // N_BATCH == c_iota // N_LAT, erep, 0.0)
    logit = lax.dot_general(ebig, l8, (((1,), (0,)), ((), ())),
                            preferred_element_type=jnp.float32)
    s = jax.nn.sigmoid(logit) * MU
    sig_ref[0] = jnp.transpose(s, (1, 0))


def _bandwidths(encoded, bandwidth_layers):
    return pl.pallas_call(
        _bw_body,
        grid=(N_LAT // _ICHUNK,),
        in_specs=[
            pl.BlockSpec((N_BATCH, N_LAT), lambda i: (0, 0)),
            pl.BlockSpec((_ICHUNK, N_LAT, N_NODES), lambda i: (i, 0, 0)),
        ],
        out_specs=pl.BlockSpec((1, N_NODES, _ICHUNK * N_BATCH),
                               lambda i: (i, 0, 0)),
        out_shape=jax.ShapeDtypeStruct((N_LAT // _ICHUNK, N_NODES,
                                        _ICHUNK * N_BATCH), jnp.float32),
    )(encoded, bandwidth_layers)


def _gather_body(table_hbm, idx_hbm, out_hbm, idx_v, rows_v, sem):
    wid = lax.axis_index("s") * 2 + lax.axis_index("c")
    base = wid * _SC_ROWS
    for c in range(_SC_ROWS // _SC_CHUNK):
        off = base + c * _SC_CHUNK
        pltpu.sync_copy(idx_hbm.at[pl.ds(off, _SC_CHUNK)], idx_v)
        pltpu.async_copy(table_hbm.at[idx_v], rows_v, sem).wait()
        pltpu.sync_copy(rows_v, out_hbm.at[pl.ds(off, _SC_CHUNK)])


def _gather(table, idx):
    mesh = plsc.VectorSubcoreMesh(core_axis_name="c", subcore_axis_name="s")
    kfn = functools.partial(
        pl.kernel, mesh=mesh,
        out_type=jax.ShapeDtypeStruct((N_NODES * MU, N_LAT), jnp.float32),
        scratch_types=[
            pltpu.VMEM((_SC_CHUNK,), jnp.int32),
            pltpu.VMEM((_SC_CHUNK, N_LAT), jnp.float32),
            pltpu.SemaphoreType.DMA,
        ],
        compiler_params=pltpu.CompilerParams(use_tc_tiling_on_sc=False),
    )(_gather_body)
    return kfn(table, idx)


def _gred(x):
    x = x[:, :256] + x[:, 256:]
    x = x[:, :128] + x[:, 128:]
    x = x[:, :64] + x[:, 64:]
    return x[:, :32] + x[:, 32:]


def _main_body(enc_ref, sig_ref, g_ref, out_ref):
    E = enc_ref[...]
    G = g_ref[...]

    nchunk = N_LAT // _ICHUNK
    s128 = jnp.concatenate([sig_ref[c] for c in range(nchunk)], axis=1)
    r_iota = lax.broadcasted_iota(jnp.int32, (128, 128), 0)
    c_iota = lax.broadcasted_iota(jnp.int32, (128, 128), 1)
    perm = jnp.where(c_iota == (r_iota % N_BATCH) * N_LAT
                     + _ICHUNK * (r_iota // 32) + (r_iota // N_BATCH) % _ICHUNK,
                     1.0, 0.0)
    sp = lax.dot_general(s128, perm, (((1,), (0,)), ((), ())),
                         preferred_element_type=jnp.float32)

    jv = (lax.broadcasted_iota(jnp.int32, (BLK, LANES), 1) // N_LAT
          ).astype(jnp.float32)
    jsq = jv * jv

    cols = []
    for b in range(N_BATCH):
        s = sp[:, b * N_LAT:(b + 1) * N_LAT]
        srep = jnp.concatenate([s] * MU, axis=1)
        win = jnp.maximum(1.0 - jsq / (srep * srep), 0.0)
        num = _gred(win * G)
        p = jnp.minimum(jnp.floor(s) + 1.0, 16.0)
        den = p - ((p - 1.0) * p * (2.0 * p - 1.0) / 6.0) / (s * s)
        contrib = (num / den) * E[b:b + 1, :]
        cols.append(jnp.sum(contrib, axis=1, keepdims=True))
    out_ref[...] = jnp.concatenate(cols, axis=1)


def _main(encoded, sig, g2):
    return pl.pallas_call(
        _main_body,
        grid=(N_NODES // BLK,),
        in_specs=[
            pl.BlockSpec((N_BATCH, N_LAT), lambda j: (0, 0)),
            pl.BlockSpec((N_LAT // _ICHUNK, BLK, _ICHUNK * N_BATCH),
                         lambda j: (0, j, 0)),
            pl.BlockSpec((BLK, LANES), lambda j: (j, 0)),
        ],
        out_specs=pl.BlockSpec((BLK, N_BATCH), lambda j: (j, 0)),
        out_shape=jax.ShapeDtypeStruct((N_NODES, N_BATCH), jnp.float32),
    )(encoded, sig, g2)


def kernel(x, neighbours, enc_W, enc_b, decoder, bandwidth_layers):
    encoded = _encode(x, enc_W, enc_b.reshape(1, N_LAT))
    sig = _bandwidths(encoded, bandwidth_layers)
    table = decoder.T
    idx = neighbours.reshape(N_NODES * MU)
    gathered = _gather(table, idx)
    g2 = gathered.reshape(N_NODES, LANES)
    out_t = _main(encoded, sig, g2)
    return out_t.T


# speedup vs baseline: 3.7471x; 1.0449x over previous
import functools

import jax
import jax.numpy as jnp
from jax import lax
from jax.experimental import pallas as pl
from jax.experimental.pallas import tpu as pltpu
from jax.experimental.pallas import tpu_sc as plsc

N_NODES = 10000
N_LAT = 32
MU = 16
N_BATCH = 4

BLK = 2000
LANES = MU * N_LAT

_SC_WORKERS = 32
_SC_ROWS = N_NODES * MU // _SC_WORKERS
_SC_CHUNK = 1000


def _encode_body(x_ref, w_ref, b_ref, out_ref):
    enc = lax.dot_general(x_ref[...], w_ref[...],
                          (((1,), (1,)), ((), ())),
                          preferred_element_type=jnp.float32)
    out_ref[...] = enc + b_ref[...]


def _encode(x, enc_W, enc_b2):
    return pl.pallas_call(
        _encode_body,
        out_shape=jax.ShapeDtypeStruct((N_BATCH, N_LAT), jnp.float32),
    )(x, enc_W, enc_b2)


_ICHUNK = 8


def _bw_body(enc_ref, l_ref, sig_ref):
    E = enc_ref[...]
    l8 = l_ref[...].reshape(_ICHUNK * N_LAT, N_NODES)
    nrow = _ICHUNK * N_BATCH
    ncol = _ICHUNK * N_LAT
    erep = jnp.concatenate([E] * _ICHUNK, axis=0)
    erep = jnp.concatenate([erep] * _ICHUNK, axis=1)
    r_iota = lax.broadcasted_iota(jnp.int32, (nrow, ncol), 0)
    c_iota = lax.broadcasted_iota(jnp.int32, (nrow, ncol), 1)
    ebig = jnp.where(r_iota // N_BATCH == c_iota // N_LAT, erep, 0.0)
    logit = lax.dot_general(ebig, l8, (((1,), (0,)), ((), ())),
                            preferred_element_type=jnp.float32)
    s = jax.nn.sigmoid(logit) * MU
    sig_ref[0] = jnp.transpose(s, (1, 0))


def _bandwidths(encoded, bandwidth_layers):
    return pl.pallas_call(
        _bw_body,
        grid=(N_LAT // _ICHUNK,),
        in_specs=[
            pl.BlockSpec((N_BATCH, N_LAT), lambda i: (0, 0)),
            pl.BlockSpec((_ICHUNK, N_LAT, N_NODES), lambda i: (i, 0, 0)),
        ],
        out_specs=pl.BlockSpec((1, N_NODES, _ICHUNK * N_BATCH),
                               lambda i: (i, 0, 0)),
        out_shape=jax.ShapeDtypeStruct((N_LAT // _ICHUNK, N_NODES,
                                        _ICHUNK * N_BATCH), jnp.float32),
    )(encoded, bandwidth_layers)


def _gather_body(table_hbm, idx_hbm, out_hbm, idx_v, rows_v, sem):
    wid = lax.axis_index("s") * 2 + lax.axis_index("c")
    base = wid * _SC_ROWS
    for c in range(_SC_ROWS // _SC_CHUNK):
        off = base + c * _SC_CHUNK
        pltpu.sync_copy(idx_hbm.at[pl.ds(off, _SC_CHUNK)], idx_v)
        pltpu.async_copy(table_hbm.at[idx_v], rows_v, sem).wait()
        pltpu.sync_copy(rows_v, out_hbm.at[pl.ds(off, _SC_CHUNK)])


def _gather(table, idx):
    mesh = plsc.VectorSubcoreMesh(core_axis_name="c", subcore_axis_name="s")
    kfn = functools.partial(
        pl.kernel, mesh=mesh,
        out_type=jax.ShapeDtypeStruct((N_NODES * MU, N_LAT), jnp.float32),
        scratch_types=[
            pltpu.VMEM((_SC_CHUNK,), jnp.int32),
            pltpu.VMEM((_SC_CHUNK, N_LAT), jnp.float32),
            pltpu.SemaphoreType.DMA,
        ],
        compiler_params=pltpu.CompilerParams(use_tc_tiling_on_sc=False),
    )(_gather_body)
    return kfn(table, idx)


def _gred(x):
    x = x[:, :256] + x[:, 256:]
    x = x[:, :128] + x[:, 128:]
    x = x[:, :64] + x[:, 64:]
    return x[:, :32] + x[:, 32:]


def _main_body(enc_ref, sig_ref, g_ref, out_ref):
    E = enc_ref[...]
    G = g_ref[...]

    nchunk = N_LAT // _ICHUNK
    s128 = jnp.concatenate([sig_ref[c] for c in range(nchunk)], axis=1)
    r_iota = lax.broadcasted_iota(jnp.int32, (128, 128), 0)
    c_iota = lax.broadcasted_iota(jnp.int32, (128, 128), 1)
    perm = jnp.where(c_iota == (r_iota % N_BATCH) * N_LAT
                     + _ICHUNK * (r_iota // 32) + (r_iota // N_BATCH) % _ICHUNK,
                     1.0, 0.0)
    sp = lax.dot_general(s128, perm, (((1,), (0,)), ((), ())),
                         preferred_element_type=jnp.float32)

    jv = (lax.broadcasted_iota(jnp.int32, (1, LANES), 1) // N_LAT
          ).astype(jnp.float32)
    jsq = jv * jv
    sr_iota = lax.broadcasted_iota(jnp.int32, (LANES, N_LAT), 0)
    sc_iota = lax.broadcasted_iota(jnp.int32, (LANES, N_LAT), 1)
    sel = jnp.where(sr_iota % N_LAT == sc_iota, 1.0, 0.0)

    cols = []
    for b in range(N_BATCH):
        s = sp[:, b * N_LAT:(b + 1) * N_LAT]
        srep = jnp.concatenate([s] * MU, axis=1)
        win = jnp.maximum(1.0 - jsq / (srep * srep), 0.0)
        num = lax.dot_general(win * G, sel, (((1,), (0,)), ((), ())),
                              preferred_element_type=jnp.float32)
        p = jnp.minimum(jnp.floor(s) + 1.0, 16.0)
        den = p - ((p - 1.0) * p * (2.0 * p - 1.0) / 6.0) / (s * s)
        contrib = (num / den) * E[b:b + 1, :]
        cols.append(jnp.sum(contrib, axis=1, keepdims=True))
    out_ref[...] = jnp.concatenate(cols, axis=1)


def _main(encoded, sig, g2):
    return pl.pallas_call(
        _main_body,
        grid=(N_NODES // BLK,),
        in_specs=[
            pl.BlockSpec((N_BATCH, N_LAT), lambda j: (0, 0)),
            pl.BlockSpec((N_LAT // _ICHUNK, BLK, _ICHUNK * N_BATCH),
                         lambda j: (0, j, 0)),
            pl.BlockSpec((BLK, LANES), lambda j: (j, 0)),
        ],
        out_specs=pl.BlockSpec((BLK, N_BATCH), lambda j: (j, 0)),
        out_shape=jax.ShapeDtypeStruct((N_NODES, N_BATCH), jnp.float32),
    )(encoded, sig, g2)


def kernel(x, neighbours, enc_W, enc_b, decoder, bandwidth_layers):
    encoded = _encode(x, enc_W, enc_b.reshape(1, N_LAT))
    sig = _bandwidths(encoded, bandwidth_layers)
    table = decoder.T
    idx = neighbours.reshape(N_NODES * MU)
    gathered = _gather(table, idx)
    g2 = gathered.reshape(N_NODES, LANES)
    out_t = _main(encoded, sig, g2)
    return out_t.T


# speedup vs baseline: 3.7518x; 1.0012x over previous
import functools

import jax
import jax.numpy as jnp
from jax import lax
from jax.experimental import pallas as pl
from jax.experimental.pallas import tpu as pltpu
from jax.experimental.pallas import tpu_sc as plsc

N_NODES = 10000
N_LAT = 32
MU = 16
N_BATCH = 4

BLK = 2000
LANES = MU * N_LAT

_SC_WORKERS = 32
_SC_ROWS = N_NODES * MU // _SC_WORKERS
_SC_CHUNK = 1000


def _encode_body(x_ref, w_ref, b_ref, out_ref):
    enc = lax.dot_general(x_ref[...], w_ref[...],
                          (((1,), (1,)), ((), ())),
                          preferred_element_type=jnp.float32)
    out_ref[...] = enc + b_ref[...]


def _encode(x, enc_W, enc_b2):
    return pl.pallas_call(
        _encode_body,
        out_shape=jax.ShapeDtypeStruct((N_BATCH, N_LAT), jnp.float32),
    )(x, enc_W, enc_b2)


_ICHUNK = 8


def _bw_body(enc_ref, l_ref, sig_ref):
    E = enc_ref[...]
    l8 = l_ref[...].reshape(_ICHUNK * N_LAT, N_NODES)
    nrow = _ICHUNK * N_BATCH
    ncol = _ICHUNK * N_LAT
    erep = jnp.concatenate([E] * _ICHUNK, axis=0)
    erep = jnp.concatenate([erep] * _ICHUNK, axis=1)
    r_iota = lax.broadcasted_iota(jnp.int32, (nrow, ncol), 0)
    c_iota = lax.broadcasted_iota(jnp.int32, (nrow, ncol), 1)
    ebig = jnp.where(r_iota // N_BATCH == c_iota // N_LAT, erep, 0.0)
    logit = lax.dot_general(ebig, l8, (((1,), (0,)), ((), ())),
                            preferred_element_type=jnp.float32)
    s = jax.nn.sigmoid(logit) * MU
    sig_ref[0] = jnp.transpose(s, (1, 0))


def _bandwidths(encoded, bandwidth_layers):
    return pl.pallas_call(
        _bw_body,
        grid=(N_LAT // _ICHUNK,),
        in_specs=[
            pl.BlockSpec((N_BATCH, N_LAT), lambda i: (0, 0)),
            pl.BlockSpec((_ICHUNK, N_LAT, N_NODES), lambda i: (i, 0, 0)),
        ],
        out_specs=pl.BlockSpec((1, N_NODES, _ICHUNK * N_BATCH),
                               lambda i: (i, 0, 0)),
        out_shape=jax.ShapeDtypeStruct((N_LAT // _ICHUNK, N_NODES,
                                        _ICHUNK * N_BATCH), jnp.float32),
    )(encoded, bandwidth_layers)


def _gather_body(table_hbm, idx_hbm, out_hbm, idx_v, rows_v, sem):
    wid = lax.axis_index("s") * 2 + lax.axis_index("c")
    base = wid * _SC_ROWS
    for c in range(_SC_ROWS // _SC_CHUNK):
        off = base + c * _SC_CHUNK
        pltpu.sync_copy(idx_hbm.at[pl.ds(off, _SC_CHUNK)], idx_v)
        pltpu.async_copy(table_hbm.at[idx_v], rows_v, sem).wait()
        pltpu.sync_copy(rows_v, out_hbm.at[pl.ds(off, _SC_CHUNK)])


def _gather(table, idx):
    mesh = plsc.VectorSubcoreMesh(core_axis_name="c", subcore_axis_name="s")
    kfn = functools.partial(
        pl.kernel, mesh=mesh,
        out_type=jax.ShapeDtypeStruct((N_NODES * MU, N_LAT), jnp.float32),
        scratch_types=[
            pltpu.VMEM((_SC_CHUNK,), jnp.int32),
            pltpu.VMEM((_SC_CHUNK, N_LAT), jnp.float32),
            pltpu.SemaphoreType.DMA,
        ],
        compiler_params=pltpu.CompilerParams(use_tc_tiling_on_sc=False),
    )(_gather_body)
    return kfn(table, idx)


def _main_body(enc_ref, sig_ref, g_ref, out_ref):
    E = enc_ref[...]
    G = g_ref[...]

    nchunk = N_LAT // _ICHUNK
    s128 = jnp.concatenate([sig_ref[c] for c in range(nchunk)], axis=1)
    r_iota = lax.broadcasted_iota(jnp.int32, (128, 128), 0)
    c_iota = lax.broadcasted_iota(jnp.int32, (128, 128), 1)
    perm = jnp.where(c_iota == (r_iota % N_BATCH) * N_LAT
                     + _ICHUNK * (r_iota // 32) + (r_iota // N_BATCH) % _ICHUNK,
                     1.0, 0.0)
    sp = lax.dot_general(s128, perm, (((1,), (0,)), ((), ())),
                         preferred_element_type=jnp.float32)

    jv = (lax.broadcasted_iota(jnp.int32, (1, LANES), 1) // N_LAT
          ).astype(jnp.float32)
    jsq = jv * jv
    sr_iota = lax.broadcasted_iota(jnp.int32, (LANES, N_LAT), 0)
    sc_iota = lax.broadcasted_iota(jnp.int32, (LANES, N_LAT), 1)
    sel = jnp.where(sr_iota % N_LAT == sc_iota, 1.0, 0.0)

    cols = []
    for b in range(N_BATCH):
        s = sp[:, b * N_LAT:(b + 1) * N_LAT]
        srep = jnp.concatenate([s] * MU, axis=1)
        win = jnp.maximum(1.0 - jsq / (srep * srep), 0.0)
        num = lax.dot_general(win * G, sel, (((1,), (0,)), ((), ())),
                              preferred_element_type=jnp.float32)
        p = jnp.minimum(jnp.floor(s) + 1.0, 16.0)
        den = p - ((p - 1.0) * p * (2.0 * p - 1.0) / 6.0) / (s * s)
        contrib = (num / den) * E[b:b + 1, :]
        cols.append(jnp.sum(contrib, axis=1, keepdims=True))
    out_ref[...] = jnp.concatenate(cols, axis=1)


def _main(encoded, sig, g2):
    return pl.pallas_call(
        _main_body,
        grid=(N_NODES // BLK,),
        in_specs=[
            pl.BlockSpec((N_BATCH, N_LAT), lambda j: (0, 0)),
            pl.BlockSpec((N_LAT // _ICHUNK, BLK, _ICHUNK * N_BATCH),
                         lambda j: (0, j, 0)),
            pl.BlockSpec((BLK, LANES), lambda j: (j, 0)),
        ],
        out_specs=pl.BlockSpec((BLK, N_BATCH), lambda j: (j, 0)),
        out_shape=jax.ShapeDtypeStruct((N_NODES, N_BATCH), jnp.float32),
    )(encoded, sig, g2)


def kernel(x, neighbours, enc_W, enc_b, decoder, bandwidth_layers):
    encoded = _encode(x, enc_W, enc_b.reshape(1, N_LAT))
    sig = _bandwidths(encoded, bandwidth_layers)
    table = decoder.T
    idx = neighbours.reshape(N_NODES * MU)
    gathered = _gather(table, idx)
    g2 = gathered.reshape(N_NODES, LANES)
    out_t = _main(encoded, sig, g2)
    return out_t.T
